# Initial kernel scaffold; baseline (speedup 1.0000x reference)
import jax
import jax.numpy as jnp
from jax.experimental import pallas as pl


def kernel(x, edge_index, W1, b1, W2, b2, W3, b3):
    raise NotImplementedError("write your pallas kernel here")


# speedup vs baseline: 2.0257x; 2.0257x over previous
import functools

import jax
import jax.numpy as jnp
from jax import lax
from jax.experimental import pallas as pl
from jax.experimental.pallas import tpu as pltpu
from jax.experimental.pallas import tpu_sc as plsc

N = 10000
E = 160000
DC = 144
NACC = 10112
SUBROWS = NACC // 16
KB = 80
EPS = E // 16
NB = EPS // KB

_MESH = plsc.VectorSubcoreMesh(
    core_axis_name="c", subcore_axis_name="s", num_cores=2, num_subcores=16
)


def _make_prop(C):
    c_per_sc = C // 2

    @functools.partial(
        pl.kernel,
        out_type=jax.ShapeDtypeStruct((NACC, C, DC), jnp.float32),
        mesh=_MESH,
        scratch_types=[
            pltpu.VMEM((EPS,), jnp.int32),
            pltpu.VMEM((NB, KB), jnp.int32),
            pltpu.VMEM((KB,), jnp.int32),
            pltpu.VMEM((KB, DC), jnp.float32),
            pltpu.VMEM_SHARED((NACC, DC), jnp.float32),
            pltpu.SemaphoreType.DMA,
        ],
        compiler_params=pltpu.CompilerParams(use_tc_tiling_on_sc=False),
    )
    def prop(zflat, srcr, dstr, zeros, out, src_v, dst_v, sidx, rows, acc, gsem):
        c = lax.axis_index("c")
        s = lax.axis_index("s")
        pltpu.sync_copy(srcr.at[s], src_v)
        pltpu.sync_copy(dstr.at[s], dst_v)
        for ci in range(c_per_sc):
            chunk = c * c_per_sc + ci
            pltpu.sync_copy(zeros, acc.at[pl.ds(s * SUBROWS, SUBROWS)])
            plsc.subcore_barrier()

            def body(b, _, chunk=chunk):
                for j in range(KB // 16):
                    ids = src_v[pl.ds(b * KB + j * 16, 16)]
                    sidx[pl.ds(j * 16, 16)] = ids * C + chunk
                pltpu.async_copy(zflat.at[sidx], rows, gsem).wait()
                pltpu.sync_copy(rows, acc.at[dst_v.at[b]], add=True)
                return 0

            lax.fori_loop(0, NB, body, 0)
            plsc.subcore_barrier()
            pltpu.sync_copy(
                acc.at[pl.ds(s * SUBROWS, SUBROWS)],
                out.at[pl.ds(s * SUBROWS, SUBROWS), chunk],
            )

    return prop


_PROP = {2: _make_prop(2), 4: _make_prop(4), 8: _make_prop(8)}


def _matmul_bias_relu(x, w, b):
    m, k = x.shape
    n = w.shape[1]
    bm = 400
    bk = 384 if k % 384 == 0 else k
    bn = 384 if n % 384 == 0 else n
    grid = (m // bm, n // bn, k // bk)
    nk = grid[2]

    def mm(x_ref, w_ref, b_ref, o_ref, acc_ref):
        kk = pl.program_id(2)

        @pl.when(kk == 0)
        def _():
            acc_ref[...] = jnp.zeros_like(acc_ref)

        acc_ref[...] += jnp.dot(
            x_ref[...], w_ref[...], preferred_element_type=jnp.float32
        )

        @pl.when(kk == nk - 1)
        def _():
            o_ref[...] = jnp.maximum(acc_ref[...] + b_ref[...], 0.0)

    return pl.pallas_call(
        mm,
        grid=grid,
        in_specs=[
            pl.BlockSpec((bm, bk), lambda i, j, kk: (i, kk)),
            pl.BlockSpec((bk, bn), lambda i, j, kk: (kk, j)),
            pl.BlockSpec((1, bn), lambda i, j, kk: (0, j)),
        ],
        out_specs=pl.BlockSpec((bm, bn), lambda i, j, kk: (i, j)),
        out_shape=jax.ShapeDtypeStruct((m, n), jnp.float32),
        scratch_shapes=[pltpu.VMEM((bm, bn), jnp.float32)],
        compiler_params=pltpu.CompilerParams(
            dimension_semantics=("parallel", "parallel", "arbitrary")
        ),
    )(x, w, b.reshape(1, -1))


def _cheb_layer(h, dis, srcr, dstr, zeros, Ws, bias):
    K, D, _ = Ws.shape
    C = D // DC
    prop = _PROP[C]

    def do_prop(t):
        zflat = (dis[:, None] * t).reshape(N * C, DC)
        mc = prop(zflat, srcr, dstr, zeros)
        return mc[:N].reshape(N, D)

    terms = [h]
    tx1 = -dis[:, None] * do_prop(h)
    terms.append(tx1)
    tx_prev, tx_pp = tx1, h
    for _ in range(2, K):
        tx = -2.0 * dis[:, None] * do_prop(tx_prev) - tx_pp
        terms.append(tx)
        tx_pp, tx_prev = tx_prev, tx
    xcat = jnp.concatenate(terms, axis=1)
    wcat = Ws.reshape(K * D, -1)
    return _matmul_bias_relu(xcat, wcat, bias)


def kernel(x, edge_index, W1, b1, W2, b2, W3, b3):
    src = edge_index[0]
    dst = edge_index[1]
    mask = src != dst
    trash = jnp.int32(N)
    src2 = jnp.where(mask, src, trash)
    dst2 = jnp.where(mask, dst, trash)
    zeros = jnp.zeros((SUBROWS, DC), jnp.float32)

    ones_flat = jnp.ones((N * 2, DC), jnp.float32)
    degc = _PROP[2](
        ones_flat,
        dst.reshape(16, EPS),
        src2.reshape(16, NB, KB),
        zeros,
    )
    deg = degc[:N, 0, 0]
    dis = jnp.where(deg > 0, lax.rsqrt(jnp.maximum(deg, 1.0)), 0.0)

    srcr = src.reshape(16, EPS)
    dstr = dst2.reshape(16, NB, KB)

    h = _cheb_layer(x, dis, srcr, dstr, zeros, W1, b1)
    h = _cheb_layer(h, dis, srcr, dstr, zeros, W2, b2)
    h = _cheb_layer(h, dis, srcr, dstr, zeros, W3, b3)
    return h


# speedup vs baseline: 2.1467x; 1.0597x over previous
import functools

import jax
import jax.numpy as jnp
from jax import lax
from jax.experimental import pallas as pl
from jax.experimental.pallas import tpu as pltpu
from jax.experimental.pallas import tpu_sc as plsc

N = 10000
E = 160000
DC = 144
NACC = 10112
SUBROWS = NACC // 16
KB = 80
EPS = E // 16
NB = EPS // KB

_MESH = plsc.VectorSubcoreMesh(
    core_axis_name="c", subcore_axis_name="s", num_cores=2, num_subcores=16
)


def _make_prop(C):
    c_per_sc = C // 2

    @functools.partial(
        pl.kernel,
        out_type=jax.ShapeDtypeStruct((NACC, C, DC), jnp.float32),
        mesh=_MESH,
        scratch_types=[
            pltpu.VMEM((NB, KB), jnp.int32),
            pltpu.VMEM((KB,), jnp.int32),
            pltpu.VMEM((KB,), jnp.int32),
            pltpu.VMEM((KB, DC), jnp.float32),
            pltpu.VMEM((KB, DC), jnp.float32),
            pltpu.VMEM_SHARED((NACC, DC), jnp.float32),
            pltpu.SemaphoreType.DMA,
            pltpu.SemaphoreType.DMA,
            pltpu.SemaphoreType.DMA,
            pltpu.SemaphoreType.DMA,
            pltpu.SemaphoreType.DMA,
            pltpu.SemaphoreType.DMA,
        ],
        compiler_params=pltpu.CompilerParams(use_tc_tiling_on_sc=False),
    )
    def prop(
        zflat, gidx, dstr, zeros, out,
        dst_v, sidx0, sidx1, rows0, rows1, acc,
        isem0, isem1, gsem0, gsem1, ssem0, ssem1,
    ):
        c = lax.axis_index("c")
        s = lax.axis_index("s")
        pltpu.sync_copy(dstr.at[s], dst_v)
        for ci in range(c_per_sc):
            chunk = c * c_per_sc + ci
            base = (chunk * 16 + s) * NB
            pltpu.sync_copy(zeros, acc.at[pl.ds(s * SUBROWS, SUBROWS)])
            plsc.subcore_barrier()

            def body(u, _, base=base):
                b0 = 2 * u
                b1 = b0 + 1
                i0 = pltpu.async_copy(gidx.at[base + b0], sidx0, isem0)
                i1 = pltpu.async_copy(gidx.at[base + b1], sidx1, isem1)
                i0.wait()
                g0 = pltpu.async_copy(zflat.at[sidx0], rows0, gsem0)
                i1.wait()
                g1 = pltpu.async_copy(zflat.at[sidx1], rows1, gsem1)
                g0.wait()
                s0 = pltpu.async_copy(rows0, acc.at[dst_v.at[b0]], ssem0, add=True)
                g1.wait()
                s1 = pltpu.async_copy(rows1, acc.at[dst_v.at[b1]], ssem1, add=True)
                s0.wait()
                s1.wait()
                return 0

            lax.fori_loop(0, NB // 2, body, 0)
            pltpu.async_copy(gidx.at[base + NB - 1], sidx0, isem0).wait()
            pltpu.async_copy(zflat.at[sidx0], rows0, gsem0).wait()
            pltpu.sync_copy(rows0, acc.at[dst_v.at[NB - 1]], add=True)
            plsc.subcore_barrier()
            pltpu.sync_copy(
                acc.at[pl.ds(s * SUBROWS, SUBROWS)],
                out.at[pl.ds(s * SUBROWS, SUBROWS), chunk],
            )

    return prop


_PROP = {2: _make_prop(2), 4: _make_prop(4), 8: _make_prop(8)}


def _matmul_bias_relu(x, w, b):
    m, k = x.shape
    n = w.shape[1]
    bm = 400
    bk = 384 if k % 384 == 0 else k
    bn = 384 if n % 384 == 0 else n
    grid = (m // bm, n // bn, k // bk)
    nk = grid[2]

    def mm(x_ref, w_ref, b_ref, o_ref, acc_ref):
        kk = pl.program_id(2)

        @pl.when(kk == 0)
        def _():
            acc_ref[...] = jnp.zeros_like(acc_ref)

        acc_ref[...] += jnp.dot(
            x_ref[...], w_ref[...], preferred_element_type=jnp.float32
        )

        @pl.when(kk == nk - 1)
        def _():
            o_ref[...] = jnp.maximum(acc_ref[...] + b_ref[...], 0.0)

    return pl.pallas_call(
        mm,
        grid=grid,
        in_specs=[
            pl.BlockSpec((bm, bk), lambda i, j, kk: (i, kk)),
            pl.BlockSpec((bk, bn), lambda i, j, kk: (kk, j)),
            pl.BlockSpec((1, bn), lambda i, j, kk: (0, j)),
        ],
        out_specs=pl.BlockSpec((bm, bn), lambda i, j, kk: (i, j)),
        out_shape=jax.ShapeDtypeStruct((m, n), jnp.float32),
        scratch_shapes=[pltpu.VMEM((bm, bn), jnp.float32)],
        compiler_params=pltpu.CompilerParams(
            dimension_semantics=("parallel", "parallel", "arbitrary")
        ),
    )(x, w, b.reshape(1, -1))


def _gather_ids(ids, C):
    g = ids[None, :] * C + jnp.arange(C, dtype=jnp.int32)[:, None]
    return g.reshape(C * 16 * NB, KB)


def _cheb_layer(h, dis, gidx, dstr, zeros, Ws, bias):
    K, D, _ = Ws.shape
    C = D // DC
    prop = _PROP[C]

    def do_prop(t):
        zflat = (dis[:, None] * t).reshape(N * C, DC)
        mc = prop(zflat, gidx, dstr, zeros)
        return mc[:N].reshape(N, D)

    terms = [h]
    tx1 = -dis[:, None] * do_prop(h)
    terms.append(tx1)
    tx_prev, tx_pp = tx1, h
    for _ in range(2, K):
        tx = -2.0 * dis[:, None] * do_prop(tx_prev) - tx_pp
        terms.append(tx)
        tx_pp, tx_prev = tx_prev, tx
    xcat = jnp.concatenate(terms, axis=1)
    wcat = Ws.reshape(K * D, -1)
    return _matmul_bias_relu(xcat, wcat, bias)


def kernel(x, edge_index, W1, b1, W2, b2, W3, b3):
    src = edge_index[0]
    dst = edge_index[1]
    mask = src != dst
    trash = jnp.int32(N)
    src2 = jnp.where(mask, src, trash)
    dst2 = jnp.where(mask, dst, trash)
    zeros = jnp.zeros((SUBROWS, DC), jnp.float32)

    ones_flat = jnp.ones((N * 2, DC), jnp.float32)
    degc = _PROP[2](
        ones_flat,
        _gather_ids(dst, 2),
        src2.reshape(16, NB, KB),
        zeros,
    )
    deg = degc[:N, 0, 0]
    dis = jnp.where(deg > 0, lax.rsqrt(jnp.maximum(deg, 1.0)), 0.0)

    dstr = dst2.reshape(16, NB, KB)
    gidx8 = _gather_ids(src, 8)
    gidx4 = _gather_ids(src, 4)

    h = _cheb_layer(x, dis, gidx8, dstr, zeros, W1, b1)
    h = _cheb_layer(h, dis, gidx8, dstr, zeros, W2, b2)
    h = _cheb_layer(h, dis, gidx4, dstr, zeros, W3, b3)
    return h


# speedup vs baseline: 2.1610x; 1.0066x over previous
import functools

import jax
import jax.numpy as jnp
from jax import lax
from jax.experimental import pallas as pl
from jax.experimental.pallas import tpu as pltpu
from jax.experimental.pallas import tpu_sc as plsc

N = 10000
E = 160000
DC = 144
NACC = 10112
SUBROWS = NACC // 16
KB = 80
EPS = E // 16
NB = EPS // KB

_MESH = plsc.VectorSubcoreMesh(
    core_axis_name="c", subcore_axis_name="s", num_cores=2, num_subcores=16
)


def _make_prop(C, dcw=DC):
    c_per_sc = C // 2

    @functools.partial(
        pl.kernel,
        out_type=jax.ShapeDtypeStruct((NACC, C, dcw), jnp.float32),
        mesh=_MESH,
        scratch_types=[
            pltpu.VMEM((2, KB), jnp.int32),
            pltpu.VMEM((2, KB), jnp.int32),
            pltpu.VMEM((KB, dcw), jnp.float32),
            pltpu.VMEM((KB, dcw), jnp.float32),
            pltpu.VMEM_SHARED((NACC, dcw), jnp.float32),
            pltpu.SemaphoreType.DMA,
            pltpu.SemaphoreType.DMA,
            pltpu.SemaphoreType.DMA,
            pltpu.SemaphoreType.DMA,
            pltpu.SemaphoreType.DMA,
            pltpu.SemaphoreType.DMA,
        ],
        compiler_params=pltpu.CompilerParams(use_tc_tiling_on_sc=False),
    )
    def prop(
        zflat, gidx, zeros, out,
        idx0, idx1, rows0, rows1, acc,
        isem0, isem1, gsem0, gsem1, ssem0, ssem1,
    ):
        c = lax.axis_index("c")
        s = lax.axis_index("s")
        dummy = zeros.at[pl.ds(0, KB)]
        for ci in range(c_per_sc):
            chunk = c * c_per_sc + ci
            base = (chunk * 16 + s) * NB
            pltpu.sync_copy(zeros, acc.at[pl.ds(s * SUBROWS, SUBROWS)])
            plsc.subcore_barrier()

            def body(u, _, base=base):
                @pl.when(u > 0)
                def _():
                    pltpu.make_async_copy(dummy, rows0, ssem0).wait()
                    pltpu.make_async_copy(dummy, rows1, ssem1).wait()

                i0 = pltpu.async_copy(gidx.at[base + 2 * u], idx0, isem0)
                i1 = pltpu.async_copy(gidx.at[base + 2 * u + 1], idx1, isem1)
                i0.wait()
                g0 = pltpu.async_copy(zflat.at[idx0.at[0]], rows0, gsem0)
                i1.wait()
                g1 = pltpu.async_copy(zflat.at[idx1.at[0]], rows1, gsem1)
                g0.wait()
                pltpu.async_copy(rows0, acc.at[idx0.at[1]], ssem0, add=True)
                g1.wait()
                pltpu.async_copy(rows1, acc.at[idx1.at[1]], ssem1, add=True)
                return 0

            lax.fori_loop(0, NB // 2, body, 0)
            pltpu.make_async_copy(dummy, rows0, ssem0).wait()
            pltpu.make_async_copy(dummy, rows1, ssem1).wait()
            pltpu.async_copy(gidx.at[base + NB - 1], idx0, isem0).wait()
            pltpu.async_copy(zflat.at[idx0.at[0]], rows0, gsem0).wait()
            pltpu.sync_copy(rows0, acc.at[idx0.at[1]], add=True)
            plsc.subcore_barrier()
            pltpu.sync_copy(
                acc.at[pl.ds(s * SUBROWS, SUBROWS)],
                out.at[pl.ds(s * SUBROWS, SUBROWS), chunk],
            )

    return prop


_PROP = {4: _make_prop(4), 8: _make_prop(8)}
_DEGPROP = _make_prop(2, dcw=16)


def _matmul_bias_relu(x, w, b):
    m, k = x.shape
    n = w.shape[1]
    bm = 400
    bk = 384 if k % 384 == 0 else k
    bn = 384 if n % 384 == 0 else n
    grid = (m // bm, n // bn, k // bk)
    nk = grid[2]

    def mm(x_ref, w_ref, b_ref, o_ref, acc_ref):
        kk = pl.program_id(2)

        @pl.when(kk == 0)
        def _():
            acc_ref[...] = jnp.zeros_like(acc_ref)

        acc_ref[...] += jnp.dot(
            x_ref[...], w_ref[...], preferred_element_type=jnp.float32
        )

        @pl.when(kk == nk - 1)
        def _():
            o_ref[...] = jnp.maximum(acc_ref[...] + b_ref[...], 0.0)

    return pl.pallas_call(
        mm,
        grid=grid,
        in_specs=[
            pl.BlockSpec((bm, bk), lambda i, j, kk: (i, kk)),
            pl.BlockSpec((bk, bn), lambda i, j, kk: (kk, j)),
            pl.BlockSpec((1, bn), lambda i, j, kk: (0, j)),
        ],
        out_specs=pl.BlockSpec((bm, bn), lambda i, j, kk: (i, j)),
        out_shape=jax.ShapeDtypeStruct((m, n), jnp.float32),
        scratch_shapes=[pltpu.VMEM((bm, bn), jnp.float32)],
        compiler_params=pltpu.CompilerParams(
            dimension_semantics=("parallel", "parallel", "arbitrary")
        ),
    )(x, w, b.reshape(1, -1))


def _edge_ids(gat, sca, C):
    g = gat[None, :] * C + jnp.arange(C, dtype=jnp.int32)[:, None]
    d = jnp.broadcast_to(sca, (C, E))
    a = jnp.stack([g, d], axis=1)
    a = a.reshape(C, 2, 16, NB, KB).transpose(0, 2, 3, 1, 4)
    return a.reshape(C * 16 * NB, 2, KB)


def _cheb_layer(h, dis, gidx, zeros, Ws, bias):
    K, D, _ = Ws.shape
    C = D // DC
    prop = _PROP[C]

    def do_prop(t):
        zflat = (dis[:, None] * t).reshape(N * C, DC)
        mc = prop(zflat, gidx, zeros)
        return mc[:N].reshape(N, D)

    terms = [h]
    tx1 = -dis[:, None] * do_prop(h)
    terms.append(tx1)
    tx_prev, tx_pp = tx1, h
    for _ in range(2, K):
        tx = -2.0 * dis[:, None] * do_prop(tx_prev) - tx_pp
        terms.append(tx)
        tx_pp, tx_prev = tx_prev, tx
    xcat = jnp.concatenate(terms, axis=1)
    wcat = Ws.reshape(K * D, -1)
    return _matmul_bias_relu(xcat, wcat, bias)


def kernel(x, edge_index, W1, b1, W2, b2, W3, b3):
    src = edge_index[0]
    dst = edge_index[1]
    mask = src != dst
    trash = jnp.int32(N)
    src2 = jnp.where(mask, src, trash)
    dst2 = jnp.where(mask, dst, trash)
    zeros = jnp.zeros((SUBROWS, DC), jnp.float32)

    degc = _DEGPROP(
        jnp.ones((N * 2, 16), jnp.float32),
        _edge_ids(dst, src2, 2),
        jnp.zeros((SUBROWS, 16), jnp.float32),
    )
    deg = degc[:N, 0, 0]
    dis = jnp.where(deg > 0, lax.rsqrt(jnp.maximum(deg, 1.0)), 0.0)

    gidx8 = _edge_ids(src, dst2, 8)
    gidx4 = _edge_ids(src, dst2, 4)

    h = _cheb_layer(x, dis, gidx8, zeros, W1, b1)
    h = _cheb_layer(h, dis, gidx8, zeros, W2, b2)
    h = _cheb_layer(h, dis, gidx4, zeros, W3, b3)
    return h
